# baseline (speedup 1.0000x reference)
import functools

import jax
import jax.numpy as jnp
from jax import lax
from jax.experimental import pallas as pl
from jax.experimental.pallas import tpu as pltpu
from jax.experimental.pallas import tpu_sc as plsc

NC = 2
NS = 16
NW = NC * NS
T = 80

F32 = jnp.float32


def _make_sc_edge(n, e_total, emit_epre):
  ew = e_total // NW
  steps = ew // T
  npt = (n // NS) // 8 * 8
  nrem = n - NS * npt
  zc = 48
  mesh = plsc.VectorSubcoreMesh(
      core_axis_name="c", subcore_axis_name="s", num_cores=NC,
      num_subcores=NS)

  out_type = [jax.ShapeDtypeStruct((NC * n, 128), jnp.bfloat16)]
  if emit_epre:
    out_type = [jax.ShapeDtypeStruct((e_total, 64), F32),
                jax.ShapeDtypeStruct((NW, 1, 128), F32)] + out_type

  scratch = [
      pltpu.VMEM((steps, 1, T), jnp.int32),
      pltpu.VMEM((steps, 1, T), jnp.int32),
      pltpu.VMEM((2, T, 64), F32),
      pltpu.VMEM((2, T, 128), F32),
      pltpu.VMEM((2, T, 64), F32),
      pltpu.VMEM((2, T, 128), jnp.bfloat16),
      pltpu.VMEM((2, T, 64), F32),
      pltpu.VMEM((zc, 128), jnp.bfloat16),
      pltpu.VMEM((1, 128), F32),
      pltpu.VMEM_SHARED((n, 128), jnp.bfloat16),
      pltpu.SemaphoreType.DMA,
      pltpu.SemaphoreType.DMA,
      pltpu.SemaphoreType.DMA,
      pltpu.SemaphoreType.DMA,
      pltpu.SemaphoreType.DMA,
  ]

  def body(ah, bvt, ce, src2, dst2, *rest):
    if emit_epre:
      epre_o, stats_o, nd_o = rest[0], rest[1], rest[2]
      rest = rest[3:]
    else:
      nd_o = rest[0]
      rest = rest[1:]
    (idx_s, idx_d, a_v, bv_v, ce_v, pay_v, epre_v, zbuf, stat_v, acc,
     sem_i0, sem_i1, sem_o0, sem_o1, sem_sc) = rest
    sem_i = (sem_i0, sem_i1)
    sem_o = (sem_o0, sem_o1)

    cid = lax.axis_index("c")
    sid = lax.axis_index("s")
    wid = cid * NS + sid

    def zrow(i, carry):
      for c in range(4):
        zbuf[i, pl.ds(c * 32, 32)] = jnp.zeros((32,), jnp.bfloat16)
      return carry
    lax.fori_loop(0, zc, zrow, 0)
    for k in range(npt // zc):
      pltpu.sync_copy(zbuf, acc.at[pl.ds(sid * npt + k * zc, zc)])
    if npt % zc:
      r = npt % zc
      pltpu.sync_copy(zbuf.at[pl.ds(0, r)],
                      acc.at[pl.ds(sid * npt + npt - r, r)])
    if nrem:
      @pl.when(sid == NS - 1)
      def _():
        pltpu.sync_copy(zbuf.at[pl.ds(0, nrem)],
                        acc.at[pl.ds(NS * npt, nrem)])
    plsc.subcore_barrier()

    pltpu.sync_copy(src2.at[wid], idx_s)
    pltpu.sync_copy(dst2.at[wid], idx_d)

    def in_copies(j, p):
      gbase = wid * ew + j * T
      return (
          pltpu.make_async_copy(ah.at[idx_d.at[j, 0]], a_v.at[p],
                                sem_i[p]),
          pltpu.make_async_copy(bvt.at[idx_s.at[j, 0]], bv_v.at[p],
                                sem_i[p]),
          pltpu.make_async_copy(ce.at[pl.ds(gbase, T)], ce_v.at[p],
                                sem_i[p]),
      )

    def out_copies(j, p):
      gbase = wid * ew + j * T
      cps = [pltpu.make_async_copy(pay_v.at[p], acc.at[idx_d.at[j, 0]],
                                   sem_o[p])]
      if emit_epre:
        cps.append(pltpu.make_async_copy(epre_v.at[p],
                                         epre_o.at[pl.ds(gbase, T)],
                                         sem_o[p]))
      return cps

    def start_in(j, p):
      for cp in in_copies(j, p):
        cp.start()

    def edge_fn(p):
      def edge(t, car):
        new = []
        for c in range(4):
          x = (a_v[p, t, pl.ds(c * 16, 16)] + bv_v[p, t, pl.ds(c * 16, 16)]
               + ce_v[p, t, pl.ds(c * 16, 16)])
          sg = 1.0 / (1.0 + jnp.exp(-x))
          vv = bv_v[p, t, pl.ds(64 + c * 16, 16)]
          pay_v[p, t, pl.ds(c * 32, 32)] = plsc.pack(
              sg * vv, sg, format=plsc.PackFormat.INTERLEAVED)
          if emit_epre:
            epre_v[p, t, pl.ds(c * 16, 16)] = x
            new.append(car[c] + x)
            new.append(car[4 + c] + x * x)
        if emit_epre:
          return tuple(new[0::2]) + tuple(new[1::2])
        return car
      return edge

    def start_out(j, p):
      cps = out_copies(j, p)
      cps[0].start(add=True)
      for cp in cps[1:]:
        cp.start()

    def epre_copy(j, p):
      gbase = wid * ew + j * T
      return pltpu.make_async_copy(epre_v.at[p], epre_o.at[pl.ds(gbase, T)],
                                   sem_o[p])

    def scatter_copy(j, p):
      return pltpu.make_async_copy(pay_v.at[p], acc.at[idx_d.at[j, 0]],
                                   sem_sc)

    def do_step(j, p, car, first, last, first_scatter=False):
      for cp in in_copies(j, p):
        cp.wait()
      if emit_epre and not first:
        epre_copy(j - 2, p).wait()
      car = plsc.parallel_loop(0, T, unroll=4, carry=car)(edge_fn(p))
      if not last:
        start_in(j + 2, p)
      if not first_scatter:
        scatter_copy(j - 1, 1 - p).wait()
      scatter_copy(j, p).start(add=True)
      if emit_epre:
        epre_copy(j, p).start()
      return car

    assert steps % 2 == 1 and steps >= 5
    init = tuple(jnp.zeros((16,), F32) for _ in range(8))
    start_in(0, 0)
    start_in(1, 1)
    car = do_step(0, 0, init, first=True, last=False, first_scatter=True)
    car = do_step(1, 1, car, first=True, last=False)

    def pair(k, car):
      j0 = 2 * k
      car = do_step(j0, 0, car, first=False, last=False)
      car = do_step(j0 + 1, 1, car, first=False, last=False)
      return car

    car = lax.fori_loop(1, (steps - 5) // 2 + 1, pair, car)
    car = do_step(steps - 3, 0, car, first=False, last=False)
    car = do_step(steps - 2, 1, car, first=False, last=True)
    stats = do_step(steps - 1, 0, car, first=False, last=True)
    scatter_copy(steps - 1, 0).wait()
    if emit_epre:
      epre_copy(steps - 2, 1).wait()
      epre_copy(steps - 1, 0).wait()

    if emit_epre:
      for c in range(4):
        stat_v[0, pl.ds(c * 16, 16)] = stats[c]
        stat_v[0, pl.ds(64 + c * 16, 16)] = stats[4 + c]
      pltpu.sync_copy(stat_v, stats_o.at[wid])

    plsc.subcore_barrier()
    pltpu.sync_copy(acc.at[pl.ds(sid * npt, npt)],
                    nd_o.at[pl.ds(cid * n + sid * npt, npt)])
    if nrem:
      @pl.when(sid == NS - 1)
      def _():
        pltpu.sync_copy(acc.at[pl.ds(NS * npt, nrem)],
                        nd_o.at[pl.ds(cid * n + NS * npt, nrem)])

  return pl.kernel(body, out_type=out_type, mesh=mesh,
                   scratch_types=scratch,
                   compiler_params=pltpu.CompilerParams(
                       use_tc_tiling_on_sc=False,
                       needs_layout_passes=False))


def _dot(a, b):
  return jnp.dot(a, b, preferred_element_type=F32)


def _k1_body(h_ref, wh, bh, wa, wb, wv, wu, bu,
             h0_ref, ah_ref, bv_ref, uh_ref):
  x = _dot(h_ref[...], wh[...]) + bh[...]
  h0_ref[...] = x
  ah_ref[...] = _dot(x, wa[...])
  bv_ref[:, :64] = _dot(x, wb[...])
  bv_ref[:, 64:] = _dot(x, wv[...])
  uh_ref[...] = _dot(x, wu[...]) + bu[...]


def _bdot(a, b):
  return jnp.dot(a.astype(jnp.bfloat16), b.astype(jnp.bfloat16),
                 preferred_element_type=F32)


def _k2_body(e_ref, we, be, wc, cb, e0_ref, ce_ref):
  x = _bdot(e_ref[...], we[...]) + be[...]
  e0_ref[...] = x
  ce_ref[...] = _bdot(x, wc[...]) + cb[...]


def _k5_body(e0_ref, ep_ref, sc_ref, sh_ref, wc, cb, ce_ref):
  en = jnp.maximum(ep_ref[...] * sc_ref[...] + sh_ref[...], 0.0)
  e1 = e0_ref[...] + en
  ce_ref[...] = _bdot(e1, wc[...]) + cb[...]


def _hupdate(hin, uh, num2, den2, n, gnh, bnh):
  num = num2[0:n, :] + num2[n:2 * n, :]
  den = den2[0:n, :] + den2[n:2 * n, :]
  hp = uh + num / (den + 1e-6)
  mu = jnp.mean(hp, axis=0, keepdims=True)
  var = jnp.mean((hp - mu) ** 2, axis=0, keepdims=True)
  hn = jnp.maximum((hp - mu) / jnp.sqrt(var + 1e-5) * gnh + bnh, 0.0)
  return hin + hn


def _make_k4(n, e_total):
  def body(hin, uh, num2, den2, es, gnh, bnh, gne, bne, wa, wb, wv, wu, bu,
           h1_ref, ah_ref, bv_ref, uh1_ref, sc_ref, sh_ref):
    h1 = _hupdate(hin[...], uh[...], num2[...], den2[...], n, gnh[...],
                  bnh[...])
    h1_ref[...] = h1
    ah_ref[...] = _dot(h1, wa[...])
    bv_ref[:, :64] = _dot(h1, wb[...])
    bv_ref[:, 64:] = _dot(h1, wv[...])
    uh1_ref[...] = _dot(h1, wu[...]) + bu[...]
    ssum = jnp.sum(es[:, 0:64], axis=0, keepdims=True)
    ssq = jnp.sum(es[:, 64:128], axis=0, keepdims=True)
    mue = ssum / e_total
    vare = ssq / e_total - mue * mue
    scale = gne[...] / jnp.sqrt(vare + 1e-5)
    sc_ref[...] = scale
    sh_ref[...] = bne[...] - mue * scale
  return body


def _make_k7(n):
  def body(hin, uh, num2, den2, gnh, bnh, wmu, bmu, wlv, blv, mu_ref, lv_ref):
    h2 = _hupdate(hin[...], uh[...], num2[...], den2[...], n, gnh[...],
                  bnh[...])
    mu_ref[...] = _dot(h2, wmu[...]) + bmu[...]
    lv_ref[...] = _dot(h2, wlv[...]) + blv[...]
  return body


def kernel(h, e, edge_index, Wh_emb, bh_emb, We_emb, be_emb, A, B, C, U, V,
           bA, bB, bC, bU, bV, gn_h, bn_h, gn_e, bn_e, Wmu, bmu, Wlv, blv):
  n, dn = h.shape
  e_total, de = e.shape
  z = Wmu.shape[1]

  steps = e_total // (NW * T)
  src2 = edge_index[0].reshape(NW, steps, 1, T)
  dst2 = edge_index[1].reshape(NW, steps, 1, T)
  r1 = lambda v: v.reshape(1, -1)
  cb0 = r1(bA[0] + bB[0] + bC[0])
  cb1 = r1(bA[1] + bB[1] + bC[1])

  h0e, ah0, bv0, uh0 = pl.pallas_call(
      _k1_body,
      out_shape=[jax.ShapeDtypeStruct((n, 64), F32),
                 jax.ShapeDtypeStruct((n, 64), F32),
                 jax.ShapeDtypeStruct((n, 128), F32),
                 jax.ShapeDtypeStruct((n, 64), F32)],
  )(h, Wh_emb, r1(bh_emb), A[0], B[0], V[0], U[0], r1(bU[0]))

  BE = 2000
  grid = (e_total // BE,)
  wspec = lambda shape: pl.BlockSpec(shape, lambda i: (0, 0))
  e0, ce0 = pl.pallas_call(
      _k2_body,
      grid=grid,
      in_specs=[pl.BlockSpec((BE, de), lambda i: (i, 0)),
                wspec((de, 64)), wspec((1, 64)), wspec((64, 64)),
                wspec((1, 64))],
      out_specs=[pl.BlockSpec((BE, 64), lambda i: (i, 0)),
                 pl.BlockSpec((BE, 64), lambda i: (i, 0))],
      out_shape=[jax.ShapeDtypeStruct((e_total, 64), F32),
                 jax.ShapeDtypeStruct((e_total, 64), F32)],
  )(e, We_emb, r1(be_emb), C[0], cb0)

  def _unpack_nd(nd):
    nd32 = nd.astype(F32).reshape(2 * n, 4, 16, 2)
    return nd32[..., 0].reshape(2 * n, 64), nd32[..., 1].reshape(2 * n, 64)

  sc0 = _make_sc_edge(n, e_total, emit_epre=True)
  epre0, est0, nd0 = sc0(ah0, bv0, ce0, src2, dst2)
  est0 = est0.reshape(NW, 128)
  num0, den0 = _unpack_nd(nd0)

  h1, ah1, bv1, uh1, sc_e, sh_e = pl.pallas_call(
      _make_k4(n, e_total),
      out_shape=[jax.ShapeDtypeStruct((n, 64), F32),
                 jax.ShapeDtypeStruct((n, 64), F32),
                 jax.ShapeDtypeStruct((n, 128), F32),
                 jax.ShapeDtypeStruct((n, 64), F32),
                 jax.ShapeDtypeStruct((1, 64), F32),
                 jax.ShapeDtypeStruct((1, 64), F32)],
  )(h0e, uh0, num0, den0, est0, r1(gn_h[0]), r1(bn_h[0]), r1(gn_e[0]),
    r1(bn_e[0]), A[1], B[1], V[1], U[1], r1(bU[1]))

  ce1 = pl.pallas_call(
      _k5_body,
      grid=grid,
      in_specs=[pl.BlockSpec((BE, 64), lambda i: (i, 0)),
                pl.BlockSpec((BE, 64), lambda i: (i, 0)),
                wspec((1, 64)), wspec((1, 64)), wspec((64, 64)),
                wspec((1, 64))],
      out_specs=pl.BlockSpec((BE, 64), lambda i: (i, 0)),
      out_shape=jax.ShapeDtypeStruct((e_total, 64), F32),
  )(e0, epre0, sc_e, sh_e, C[1], cb1)

  sc1 = _make_sc_edge(n, e_total, emit_epre=False)
  nd1 = sc1(ah1, bv1, ce1, src2, dst2)
  if isinstance(nd1, (list, tuple)):
    nd1 = nd1[0]
  num1, den1 = _unpack_nd(nd1)

  mu, lv = pl.pallas_call(
      _make_k7(n),
      out_shape=[jax.ShapeDtypeStruct((n, z), F32),
                 jax.ShapeDtypeStruct((n, z), F32)],
  )(h1, uh1, num1, den1, r1(gn_h[1]), r1(bn_h[1]), Wmu, r1(bmu), Wlv,
    r1(blv))

  return (mu, lv)


# speedup vs baseline: 1.0304x; 1.0304x over previous
import functools

import jax
import jax.numpy as jnp
from jax import lax
from jax.experimental import pallas as pl
from jax.experimental.pallas import tpu as pltpu
from jax.experimental.pallas import tpu_sc as plsc

NC = 2
NS = 16
NW = NC * NS
T = 80

F32 = jnp.float32


def _make_sc_edge(n, e_total, emit_epre):
  ew = e_total // NW
  steps = ew // T
  npt = (n // NS) // 8 * 8
  nrem = n - NS * npt
  zc = 48
  mesh = plsc.VectorSubcoreMesh(
      core_axis_name="c", subcore_axis_name="s", num_cores=NC,
      num_subcores=NS)

  out_type = [jax.ShapeDtypeStruct((NC * n, 128), jnp.bfloat16)]
  if emit_epre:
    out_type = [jax.ShapeDtypeStruct((e_total, 64), F32),
                jax.ShapeDtypeStruct((NW, 1, 128), F32)] + out_type

  scratch = [
      pltpu.VMEM((steps, 1, T), jnp.int32),
      pltpu.VMEM((steps, 1, T), jnp.int32),
      pltpu.VMEM((2, T, 64), F32),
      pltpu.VMEM((2, T, 128), F32),
      pltpu.VMEM((2, T, 64), F32),
      pltpu.VMEM((2, T, 128), jnp.bfloat16),
      pltpu.VMEM((2, T, 64), F32),
      pltpu.VMEM((zc, 128), jnp.bfloat16),
      pltpu.VMEM((1, 128), F32),
      pltpu.VMEM_SHARED((n, 128), jnp.bfloat16),
      pltpu.SemaphoreType.DMA,
      pltpu.SemaphoreType.DMA,
      pltpu.SemaphoreType.DMA,
      pltpu.SemaphoreType.DMA,
      pltpu.SemaphoreType.DMA,
  ]

  def body(ah, bvt, ce, src2, dst2, *rest):
    if emit_epre:
      epre_o, stats_o, nd_o = rest[0], rest[1], rest[2]
      rest = rest[3:]
    else:
      nd_o = rest[0]
      rest = rest[1:]
    (idx_s, idx_d, a_v, bv_v, ce_v, pay_v, epre_v, zbuf, stat_v, acc,
     sem_i0, sem_i1, sem_o0, sem_o1, sem_sc) = rest
    sem_i = (sem_i0, sem_i1)
    sem_o = (sem_o0, sem_o1)

    cid = lax.axis_index("c")
    sid = lax.axis_index("s")
    wid = cid * NS + sid

    def zrow(i, carry):
      for c in range(4):
        zbuf[i, pl.ds(c * 32, 32)] = jnp.zeros((32,), jnp.bfloat16)
      return carry
    lax.fori_loop(0, zc, zrow, 0)
    for k in range(npt // zc):
      pltpu.sync_copy(zbuf, acc.at[pl.ds(sid * npt + k * zc, zc)])
    if npt % zc:
      r = npt % zc
      pltpu.sync_copy(zbuf.at[pl.ds(0, r)],
                      acc.at[pl.ds(sid * npt + npt - r, r)])
    if nrem:
      @pl.when(sid == NS - 1)
      def _():
        pltpu.sync_copy(zbuf.at[pl.ds(0, nrem)],
                        acc.at[pl.ds(NS * npt, nrem)])
    plsc.subcore_barrier()

    pltpu.sync_copy(src2.at[wid], idx_s)
    pltpu.sync_copy(dst2.at[wid], idx_d)

    def in_copies(j, p):
      gbase = wid * ew + j * T
      return (
          pltpu.make_async_copy(ah.at[idx_d.at[j, 0]], a_v.at[p],
                                sem_i[p]),
          pltpu.make_async_copy(bvt.at[idx_s.at[j, 0]], bv_v.at[p],
                                sem_i[p]),
          pltpu.make_async_copy(ce.at[pl.ds(gbase, T)], ce_v.at[p],
                                sem_i[p]),
      )

    def out_copies(j, p):
      gbase = wid * ew + j * T
      cps = [pltpu.make_async_copy(pay_v.at[p], acc.at[idx_d.at[j, 0]],
                                   sem_o[p])]
      if emit_epre:
        cps.append(pltpu.make_async_copy(epre_v.at[p],
                                         epre_o.at[pl.ds(gbase, T)],
                                         sem_o[p]))
      return cps

    def start_in(j, p):
      for cp in in_copies(j, p):
        cp.start()

    def edge_fn(p):
      def edge(t, car):
        new = []
        for c in range(4):
          x = (a_v[p, t, pl.ds(c * 16, 16)] + bv_v[p, t, pl.ds(c * 16, 16)]
               + ce_v[p, t, pl.ds(c * 16, 16)])
          sg = 1.0 / (1.0 + jnp.exp(-x))
          vv = bv_v[p, t, pl.ds(64 + c * 16, 16)]
          pay_v[p, t, pl.ds(c * 32, 32)] = plsc.pack(
              sg * vv, sg, format=plsc.PackFormat.INTERLEAVED)
          if emit_epre:
            epre_v[p, t, pl.ds(c * 16, 16)] = x
            new.append(car[c] + x)
            new.append(car[4 + c] + x * x)
        if emit_epre:
          return tuple(new[0::2]) + tuple(new[1::2])
        return car
      return edge

    def start_out(j, p):
      cps = out_copies(j, p)
      cps[0].start(add=True)
      for cp in cps[1:]:
        cp.start()

    def epre_copy(j, p):
      gbase = wid * ew + j * T
      return pltpu.make_async_copy(epre_v.at[p], epre_o.at[pl.ds(gbase, T)],
                                   sem_o[p])

    def scatter_copy(j, p):
      return pltpu.make_async_copy(pay_v.at[p], acc.at[idx_d.at[j, 0]],
                                   sem_sc)

    def do_step(j, p, car, first, last, first_scatter=False):
      for cp in in_copies(j, p):
        cp.wait()
      if emit_epre and not first:
        epre_copy(j - 2, p).wait()
      car = plsc.parallel_loop(0, T, unroll=4, carry=car)(edge_fn(p))
      if not last:
        start_in(j + 2, p)
      if not first_scatter:
        scatter_copy(j - 1, 1 - p).wait()
      scatter_copy(j, p).start(add=True)
      if emit_epre:
        epre_copy(j, p).start()
      return car

    assert steps % 2 == 1 and steps >= 5
    init = tuple(jnp.zeros((16,), F32) for _ in range(8))
    start_in(0, 0)
    start_in(1, 1)
    car = do_step(0, 0, init, first=True, last=False, first_scatter=True)
    car = do_step(1, 1, car, first=True, last=False)

    def pair(k, car):
      j0 = 2 * k
      car = do_step(j0, 0, car, first=False, last=False)
      car = do_step(j0 + 1, 1, car, first=False, last=False)
      return car

    car = lax.fori_loop(1, (steps - 5) // 2 + 1, pair, car)
    car = do_step(steps - 3, 0, car, first=False, last=False)
    car = do_step(steps - 2, 1, car, first=False, last=True)
    stats = do_step(steps - 1, 0, car, first=False, last=True)
    scatter_copy(steps - 1, 0).wait()
    if emit_epre:
      epre_copy(steps - 2, 1).wait()
      epre_copy(steps - 1, 0).wait()

    if emit_epre:
      for c in range(4):
        stat_v[0, pl.ds(c * 16, 16)] = stats[c]
        stat_v[0, pl.ds(64 + c * 16, 16)] = stats[4 + c]
      pltpu.sync_copy(stat_v, stats_o.at[wid])

    plsc.subcore_barrier()
    pltpu.sync_copy(acc.at[pl.ds(sid * npt, npt)],
                    nd_o.at[pl.ds(cid * n + sid * npt, npt)])
    if nrem:
      @pl.when(sid == NS - 1)
      def _():
        pltpu.sync_copy(acc.at[pl.ds(NS * npt, nrem)],
                        nd_o.at[pl.ds(cid * n + NS * npt, nrem)])

  return pl.kernel(body, out_type=out_type, mesh=mesh,
                   scratch_types=scratch,
                   compiler_params=pltpu.CompilerParams(
                       use_tc_tiling_on_sc=False,
                       needs_layout_passes=False))


def _dot(a, b):
  return jnp.dot(a, b, preferred_element_type=F32)


def _k1_body(h_ref, wh, bh, wa, wb, wv, wu, bu,
             h0_ref, ah_ref, bv_ref, uh_ref):
  x = _dot(h_ref[...], wh[...]) + bh[...]
  h0_ref[...] = x
  ah_ref[...] = _dot(x, wa[...])
  bv_ref[:, :64] = _dot(x, wb[...])
  bv_ref[:, 64:] = _dot(x, wv[...])
  uh_ref[...] = _dot(x, wu[...]) + bu[...]


def _bdot(a, b):
  return jnp.dot(a.astype(jnp.bfloat16), b.astype(jnp.bfloat16),
                 preferred_element_type=F32)


def _k2_body(e_ref, we, be, wc, cb, e0_ref, ce_ref):
  x = _dot(e_ref[...], we[...]) + be[...]
  e0_ref[...] = x.astype(jnp.bfloat16)
  ce_ref[...] = _dot(x, wc[...]) + cb[...]


def _k5_body(e0_ref, ep_ref, sc_ref, sh_ref, wc, cb, ce_ref):
  en = jnp.maximum(ep_ref[...] * sc_ref[...] + sh_ref[...], 0.0)
  e1 = e0_ref[...].astype(F32) + en
  ce_ref[...] = _dot(e1, wc[...]) + cb[...]


def _hupdate(hin, uh, num2, den2, n, gnh, bnh):
  num = num2[0:n, :] + num2[n:2 * n, :]
  den = den2[0:n, :] + den2[n:2 * n, :]
  hp = uh + num / (den + 1e-6)
  mu = jnp.mean(hp, axis=0, keepdims=True)
  var = jnp.mean((hp - mu) ** 2, axis=0, keepdims=True)
  hn = jnp.maximum((hp - mu) / jnp.sqrt(var + 1e-5) * gnh + bnh, 0.0)
  return hin + hn


def _make_k4(n, e_total):
  def body(hin, uh, num2, den2, es, gnh, bnh, gne, bne, wa, wb, wv, wu, bu,
           h1_ref, ah_ref, bv_ref, uh1_ref, sc_ref, sh_ref):
    h1 = _hupdate(hin[...], uh[...], num2[...], den2[...], n, gnh[...],
                  bnh[...])
    h1_ref[...] = h1
    ah_ref[...] = _dot(h1, wa[...])
    bv_ref[:, :64] = _dot(h1, wb[...])
    bv_ref[:, 64:] = _dot(h1, wv[...])
    uh1_ref[...] = _dot(h1, wu[...]) + bu[...]
    ssum = jnp.sum(es[:, 0:64], axis=0, keepdims=True)
    ssq = jnp.sum(es[:, 64:128], axis=0, keepdims=True)
    mue = ssum / e_total
    vare = ssq / e_total - mue * mue
    scale = gne[...] / jnp.sqrt(vare + 1e-5)
    sc_ref[...] = scale
    sh_ref[...] = bne[...] - mue * scale
  return body


def _make_k7(n):
  def body(hin, uh, num2, den2, gnh, bnh, wmu, bmu, wlv, blv, mu_ref, lv_ref):
    h2 = _hupdate(hin[...], uh[...], num2[...], den2[...], n, gnh[...],
                  bnh[...])
    mu_ref[...] = _dot(h2, wmu[...]) + bmu[...]
    lv_ref[...] = _dot(h2, wlv[...]) + blv[...]
  return body


def kernel(h, e, edge_index, Wh_emb, bh_emb, We_emb, be_emb, A, B, C, U, V,
           bA, bB, bC, bU, bV, gn_h, bn_h, gn_e, bn_e, Wmu, bmu, Wlv, blv):
  n, dn = h.shape
  e_total, de = e.shape
  z = Wmu.shape[1]

  steps = e_total // (NW * T)
  src2 = edge_index[0].reshape(NW, steps, 1, T)
  dst2 = edge_index[1].reshape(NW, steps, 1, T)
  r1 = lambda v: v.reshape(1, -1)
  cb0 = r1(bA[0] + bB[0] + bC[0])
  cb1 = r1(bA[1] + bB[1] + bC[1])

  h0e, ah0, bv0, uh0 = pl.pallas_call(
      _k1_body,
      out_shape=[jax.ShapeDtypeStruct((n, 64), F32),
                 jax.ShapeDtypeStruct((n, 64), F32),
                 jax.ShapeDtypeStruct((n, 128), F32),
                 jax.ShapeDtypeStruct((n, 64), F32)],
  )(h, Wh_emb, r1(bh_emb), A[0], B[0], V[0], U[0], r1(bU[0]))

  BE = 2000
  grid = (e_total // BE,)
  wspec = lambda shape: pl.BlockSpec(shape, lambda i: (0, 0))
  e0, ce0 = pl.pallas_call(
      _k2_body,
      grid=grid,
      in_specs=[pl.BlockSpec((BE, de), lambda i: (i, 0)),
                wspec((de, 64)), wspec((1, 64)), wspec((64, 64)),
                wspec((1, 64))],
      out_specs=[pl.BlockSpec((BE, 64), lambda i: (i, 0)),
                 pl.BlockSpec((BE, 64), lambda i: (i, 0))],
      out_shape=[jax.ShapeDtypeStruct((e_total, 64), jnp.bfloat16),
                 jax.ShapeDtypeStruct((e_total, 64), F32)],
  )(e, We_emb, r1(be_emb), C[0], cb0)

  def _unpack_nd(nd):
    nd32 = nd.astype(F32).reshape(2 * n, 4, 16, 2)
    return nd32[..., 0].reshape(2 * n, 64), nd32[..., 1].reshape(2 * n, 64)

  sc0 = _make_sc_edge(n, e_total, emit_epre=True)
  epre0, est0, nd0 = sc0(ah0, bv0, ce0, src2, dst2)
  est0 = est0.reshape(NW, 128)
  num0, den0 = _unpack_nd(nd0)

  h1, ah1, bv1, uh1, sc_e, sh_e = pl.pallas_call(
      _make_k4(n, e_total),
      out_shape=[jax.ShapeDtypeStruct((n, 64), F32),
                 jax.ShapeDtypeStruct((n, 64), F32),
                 jax.ShapeDtypeStruct((n, 128), F32),
                 jax.ShapeDtypeStruct((n, 64), F32),
                 jax.ShapeDtypeStruct((1, 64), F32),
                 jax.ShapeDtypeStruct((1, 64), F32)],
  )(h0e, uh0, num0, den0, est0, r1(gn_h[0]), r1(bn_h[0]), r1(gn_e[0]),
    r1(bn_e[0]), A[1], B[1], V[1], U[1], r1(bU[1]))

  ce1 = pl.pallas_call(
      _k5_body,
      grid=grid,
      in_specs=[pl.BlockSpec((BE, 64), lambda i: (i, 0)),
                pl.BlockSpec((BE, 64), lambda i: (i, 0)),
                wspec((1, 64)), wspec((1, 64)), wspec((64, 64)),
                wspec((1, 64))],
      out_specs=pl.BlockSpec((BE, 64), lambda i: (i, 0)),
      out_shape=jax.ShapeDtypeStruct((e_total, 64), F32),
  )(e0, epre0, sc_e, sh_e, C[1], cb1)

  sc1 = _make_sc_edge(n, e_total, emit_epre=False)
  nd1 = sc1(ah1, bv1, ce1, src2, dst2)
  if isinstance(nd1, (list, tuple)):
    nd1 = nd1[0]
  num1, den1 = _unpack_nd(nd1)

  mu, lv = pl.pallas_call(
      _make_k7(n),
      out_shape=[jax.ShapeDtypeStruct((n, z), F32),
                 jax.ShapeDtypeStruct((n, z), F32)],
  )(h1, uh1, num1, den1, r1(gn_h[1]), r1(bn_h[1]), Wmu, r1(bmu), Wlv,
    r1(blv))

  return (mu, lv)
